# baseline (speedup 1.0000x reference)
import functools

import jax
import jax.numpy as jnp
from jax import lax
from jax.experimental import pallas as pl
from jax.experimental.pallas import tpu as pltpu
from jax.experimental.pallas import tpu_sc as plsc

B, S, D = 16, 1024, 32
K = 1024
N = B * S
R = 1024
GRID = N // R


def _tc_body(z_ref, cb_ref, idx_ref, en_ref, loss_ref):
    i = pl.program_id(0)

    cb = cb_ref[...]
    e2 = jnp.sum(cb * cb, axis=1, keepdims=True)
    en = cb / jnp.maximum(jnp.sqrt(e2), 1e-12)
    en_ref[...] = en
    en2 = jnp.sum(en * en, axis=1)

    zb = z_ref[0]
    z2 = jnp.sum(zb * zb, axis=1, keepdims=True)
    zn = zb / jnp.maximum(jnp.sqrt(z2), 1e-12)
    zn2 = jnp.sum(zn * zn, axis=1, keepdims=True)

    znb = zn.astype(jnp.bfloat16)
    enb = en.astype(jnp.bfloat16)
    s = lax.dot_general(znb, enb, (((1,), (1,)), ((), ())),
                        preferred_element_type=jnp.float32)
    d = (zn2 - 2.0 * s) + en2[None, :]

    T = 256
    racc = jnp.full((R, 1), jnp.inf, dtype=jnp.float32)
    dsel = jnp.zeros((R, 1), dtype=jnp.float32)
    iacc = jnp.zeros((R, 1), dtype=jnp.int32)
    kst = lax.broadcasted_iota(jnp.int32, (R, T), 1)
    for g in range(K // T):
        dt = d[:, g * T:(g + 1) * T]
        m = jnp.min(dt, axis=1, keepdims=True)
        ig = jnp.min(jnp.where(dt == m, kst + g * T, K), axis=1, keepdims=True)
        lt = m < racc
        racc = jnp.where(lt, m.astype(jnp.bfloat16).astype(jnp.float32), racc)
        dsel = jnp.where(lt, m, dsel)
        iacc = jnp.where(lt, ig, iacc)
    idx_ref[0, 0, :] = iacc[:, 0]

    block_sum = jnp.sum(dsel)

    @pl.when(i == 0)
    def _():
        loss_ref[0, 0] = 0.0

    loss_ref[0, 0] += block_sum

    @pl.when(i == GRID - 1)
    def _():
        loss_ref[0, 0] = loss_ref[0, 0] * (1.25 / (N * D))


def _tc_quantize(zflat3, codebook):
    return pl.pallas_call(
        _tc_body,
        grid=(GRID,),
        in_specs=[
            pl.BlockSpec((1, R, D), lambda i: (i, 0, 0)),
            pl.BlockSpec((K, D), lambda i: (0, 0)),
        ],
        out_specs=[
            pl.BlockSpec((1, 1, R), lambda i: (i, 0, 0)),
            pl.BlockSpec((K, D), lambda i: (0, 0)),
            pl.BlockSpec(memory_space=pltpu.SMEM),
        ],
        out_shape=[
            jax.ShapeDtypeStruct((GRID, 1, R), jnp.int32),
            jax.ShapeDtypeStruct((K, D), jnp.float32),
            jax.ShapeDtypeStruct((1, 1), jnp.float32),
        ],
    )(zflat3, codebook)


_NC, _NS = 2, 16
_NW = _NC * _NS
_BPW = N // _NW


def _sc_gather(en, idx):
    mesh = plsc.VectorSubcoreMesh(core_axis_name="c", subcore_axis_name="s")

    @functools.partial(
        pl.kernel,
        mesh=mesh,
        compiler_params=pltpu.CompilerParams(use_tc_tiling_on_sc=False),
        out_type=jax.ShapeDtypeStruct((N, D), jnp.float32),
        scratch_types=[
            pltpu.VMEM((_BPW,), jnp.int32),
            pltpu.VMEM((_BPW, D), jnp.float32),
            pltpu.SemaphoreType.DMA,
        ],
    )
    def k(en_hbm, idx_hbm, out_hbm, idx_v, rows_v, sem):
        wid = lax.axis_index("s") * _NC + lax.axis_index("c")
        base = wid * _BPW
        pltpu.sync_copy(idx_hbm.at[pl.ds(base, _BPW)], idx_v)
        pltpu.async_copy(en_hbm.at[idx_v], rows_v, sem).wait()
        pltpu.sync_copy(rows_v, out_hbm.at[pl.ds(base, _BPW)])

    return k(en, idx)


def kernel(z, codebook):
    zflat3 = z.reshape(GRID, R, D)
    idx3, en, loss = _tc_quantize(zflat3, codebook)
    idx_flat = idx3.reshape(N)
    z_q = _sc_gather(en, idx_flat).reshape(B, S, D)
    return z_q, loss[0, 0], idx3.reshape(B, S)
